# baseline (speedup 1.0000x reference)
import jax
import jax.numpy as jnp
from jax.experimental import pallas as pl
from jax.experimental.pallas import tpu as pltpu


def _pm_body(x_ref, q_ref, o_ref, acc_ref):
    h = pl.program_id(1)
    hb = x_ref.shape[1]
    x = x_ref[0]
    q = q_ref[0, 0]
    u = jax.lax.bitcast_convert_type(x, jnp.int32)
    s = (x > q[None, None, :]).astype(jnp.int32)
    u = (u & jnp.int32(-2)) | s
    x2 = jax.lax.bitcast_convert_type(u, jnp.float32)
    m = jnp.max(x2, axis=-1)
    mb = jax.lax.bitcast_convert_type(m, jnp.int32)
    res = ((mb & 1) ^ 1).T

    @pl.when(h % 2 == 0)
    def _park():
        acc_ref[...] = res

    @pl.when(h % 2 == 1)
    def _store():
        o_ref[0, 0, :, pl.ds(pl.multiple_of((h - 1) * hb, 2 * hb), 2 * hb)] = jnp.concatenate(
            [acc_ref[...], res], axis=1
        )


def kernel(input, class_qlims):
    B, H, W, C = input.shape
    HB = 64
    q3 = class_qlims.reshape(B, 1, C)
    grid = (B, H // HB)
    return pl.pallas_call(
        _pm_body,
        grid=grid,
        in_specs=[
            pl.BlockSpec((1, HB, W, C), lambda b, h: (b, h, 0, 0)),
            pl.BlockSpec((1, 1, C), lambda b, h: (b, 0, 0)),
        ],
        out_specs=pl.BlockSpec((1, 1, W, H), lambda b, h: (b, 0, 0, 0)),
        out_shape=jax.ShapeDtypeStruct((B, 1, W, H), jnp.int32),
        scratch_shapes=[pltpu.VMEM((W, HB), jnp.int32)],
        compiler_params=pltpu.CompilerParams(
            dimension_semantics=("arbitrary", "arbitrary"),
        ),
    )(input, q3)
